# Initial kernel scaffold; baseline (speedup 1.0000x reference)
import jax
import jax.numpy as jnp
from jax.experimental import pallas as pl


def kernel(legs, votes, ideal_points, yes_points, no_points, w):
    raise NotImplementedError("write your pallas kernel here")


# speedup vs baseline: 36.9468x; 36.9468x over previous
import functools

import jax
import jax.numpy as jnp
from jax import lax
from jax.experimental import pallas as pl
from jax.experimental.pallas import tpu as pltpu
from jax.experimental.pallas import tpu_sc as plsc

B = 3276800
V = 1000000
CHUNK = 4096


def _rsqrt(x):
    i = lax.bitcast_convert_type(x, jnp.int32)
    i = jnp.int32(0x5F3759DF) - (i >> 1)
    y = lax.bitcast_convert_type(i, jnp.float32)
    for _ in range(3):
        y = y * (1.5 - 0.5 * x * y * y)
    return y


def _make_impl(b_per_w):
    n_chunks = b_per_w // CHUNK
    mesh = plsc.VectorSubcoreMesh(core_axis_name="c", subcore_axis_name="s")

    @functools.partial(
        pl.kernel,
        out_type=jax.ShapeDtypeStruct((B,), jnp.float32),
        mesh=mesh,
        compiler_params=pltpu.CompilerParams(
            needs_layout_passes=False, use_tc_tiling_on_sc=False),
        scratch_types=[
            pltpu.VMEM((CHUNK,), jnp.int32),
            pltpu.VMEM((CHUNK,), jnp.int32),
            pltpu.VMEM((CHUNK,), jnp.int32),
            pltpu.VMEM((CHUNK, 8), jnp.float32),
            pltpu.VMEM((CHUNK,), jnp.float32),
            pltpu.VMEM((64,), jnp.float32),
            pltpu.VMEM((16,), jnp.float32),
            pltpu.SemaphoreType.DMA,
        ],
    )
    def impl(legs_hbm, votes_hbm, ip_hbm, tbl_hbm, w_hbm, out_hbm,
             votes_v, idx_v, legs_v, rows_v, out_v, ip_v, w_v, sem):
        info = plsc.get_sparse_core_info()
        nc = info.num_cores
        wid = lax.axis_index("s") * nc + lax.axis_index("c")

        pltpu.sync_copy(w_hbm, w_v)
        pltpu.sync_copy(ip_hbm, ip_v)

        lanes = lax.iota(jnp.int32, 16)
        zeros = jnp.zeros((16,), jnp.int32)
        wv = w_v[...]
        w0 = wv.at[zeros].get(mode="promise_in_bounds")
        w1 = wv.at[zeros + 1].get(mode="promise_in_bounds")
        w20 = w0 * w0
        w21 = w1 * w1

        for j in range(2):
            rows = lanes + 16 * j
            i0 = rows * 2
            i1 = i0 + 1
            x0 = plsc.load_gather(ip_v, [i0])
            x1 = plsc.load_gather(ip_v, [i1])
            n2 = x0 * x0 + x1 * x1
            n = n2 * _rsqrt(n2)
            scale = jnp.where(n2 > 1.0, 1.0 / (n + 1e-7), 1.0)
            plsc.store_scatter(ip_v, [i0], x0 * scale)
            plsc.store_scatter(ip_v, [i1], x1 * scale)

        def chunk_body(g, _):
            base = wid * b_per_w + g * CHUNK
            pltpu.sync_copy(votes_hbm.at[pl.ds(base, CHUNK)], votes_v)

            def shift_body(i, _):
                k0 = i * 16
                idx_v[pl.ds(k0, 16)] = votes_v[pl.ds(k0, 16)] >> 1
                return 0

            lax.fori_loop(0, CHUNK // 16, shift_body, 0, unroll=4)
            cp = pltpu.make_async_copy(tbl_hbm.at[idx_v], rows_v, sem)
            cp.start()
            pltpu.sync_copy(legs_hbm.at[pl.ds(base, CHUNK)], legs_v)
            cp.wait()

            def group_body(i, _):
                k0 = i * 16
                v = votes_v[pl.ds(k0, 16)]
                p2 = (v & 1) * 2
                lv = legs_v[pl.ds(k0, 16)]
                g0 = plsc.load_gather(ip_v, [lv * 2])
                g1 = plsc.load_gather(ip_v, [lv * 2 + 1])
                ridx = lanes + k0
                yp0 = plsc.load_gather(rows_v, [ridx, p2])
                yp1 = plsc.load_gather(rows_v, [ridx, p2 + 1])
                np0 = plsc.load_gather(rows_v, [ridx, p2 + 4])
                np1 = plsc.load_gather(rows_v, [ridx, p2 + 5])
                dy0 = g0 - yp0
                dy1 = g1 - yp1
                dn0 = g0 - np0
                dn1 = g1 - np1
                dy = w20 * (dy0 * dy0) + w21 * (dy1 * dy1)
                dn = w20 * (dn0 * dn0) + w21 * (dn1 * dn1)
                out_v[pl.ds(k0, 16)] = jnp.exp(-0.5 * dy) - jnp.exp(-0.5 * dn)
                return 0

            lax.fori_loop(0, CHUNK // 16, group_body, 0, unroll=4)
            pltpu.sync_copy(out_v, out_hbm.at[pl.ds(base, CHUNK)])
            return 0

        lax.fori_loop(0, n_chunks, chunk_body, 0)

    return impl


def kernel(legs, votes, ideal_points, yes_points, no_points, w):
    pk = jnp.concatenate(
        [yes_points.reshape(V // 2, 4), no_points.reshape(V // 2, 4)], axis=1)
    wpad = jnp.pad(w.astype(jnp.float32), (0, 14))
    ip_flat = jnp.reshape(ideal_points.astype(jnp.float32), (-1,))
    info = plsc.get_sparse_core_info()
    nw = info.num_cores * info.num_subcores
    impl = _make_impl(B // nw)
    return impl(legs.astype(jnp.int32), votes.astype(jnp.int32),
                ip_flat, pk, wpad)


# speedup vs baseline: 41.8427x; 1.1325x over previous
import functools

import jax
import jax.numpy as jnp
from jax import lax
from jax.experimental import pallas as pl
from jax.experimental.pallas import tpu as pltpu
from jax.experimental.pallas import tpu_sc as plsc

B = 3276800
V = 1000000
CHUNK = 6400


def _rsqrt(x):
    i = lax.bitcast_convert_type(x, jnp.int32)
    i = jnp.int32(0x5F3759DF) - (i >> 1)
    y = lax.bitcast_convert_type(i, jnp.float32)
    for _ in range(3):
        y = y * (1.5 - 0.5 * x * y * y)
    return y


def _make_impl(b_per_w):
    n_chunks = b_per_w // CHUNK
    mesh = plsc.VectorSubcoreMesh(core_axis_name="c", subcore_axis_name="s")

    @functools.partial(
        pl.kernel,
        out_type=jax.ShapeDtypeStruct((B,), jnp.float32),
        mesh=mesh,
        compiler_params=pltpu.CompilerParams(
            needs_layout_passes=False, use_tc_tiling_on_sc=False),
        scratch_types=[
            pltpu.VMEM((CHUNK,), jnp.int32),
            pltpu.VMEM((CHUNK,), jnp.int32),
            pltpu.VMEM((CHUNK,), jnp.int32),
            pltpu.VMEM((CHUNK, 8), jnp.float32),
            pltpu.VMEM((CHUNK, 8), jnp.float32),
            pltpu.VMEM((CHUNK,), jnp.float32),
            pltpu.VMEM((64,), jnp.float32),
            pltpu.VMEM((16,), jnp.float32),
            pltpu.SemaphoreType.DMA,
        ],
    )
    def impl(legs_hbm, votes_hbm, ip_hbm, yes_hbm, no_hbm, w_hbm, out_hbm,
             votes_v, idx_v, legs_v, yrows_v, nrows_v, out_v, ip_v, w_v, sem):
        info = plsc.get_sparse_core_info()
        nc = info.num_cores
        wid = lax.axis_index("s") * nc + lax.axis_index("c")

        pltpu.sync_copy(w_hbm, w_v)
        pltpu.sync_copy(ip_hbm, ip_v)

        lanes = lax.iota(jnp.int32, 16)
        zeros = jnp.zeros((16,), jnp.int32)
        wv = w_v[...]
        w0 = wv.at[zeros].get(mode="promise_in_bounds")
        w1 = wv.at[zeros + 1].get(mode="promise_in_bounds")
        w20 = w0 * w0
        w21 = w1 * w1

        for j in range(2):
            rows = lanes + 16 * j
            i0 = rows * 2
            i1 = i0 + 1
            x0 = plsc.load_gather(ip_v, [i0])
            x1 = plsc.load_gather(ip_v, [i1])
            n2 = x0 * x0 + x1 * x1
            n = n2 * _rsqrt(n2)
            scale = jnp.where(n2 > 1.0, 1.0 / (n + 1e-7), 1.0)
            plsc.store_scatter(ip_v, [i0], x0 * scale)
            plsc.store_scatter(ip_v, [i1], x1 * scale)

        def chunk_body(g, _):
            base = wid * b_per_w + g * CHUNK
            pltpu.sync_copy(votes_hbm.at[pl.ds(base, CHUNK)], votes_v)

            def shift_body(i, _):
                k0 = i * 16
                idx_v[pl.ds(k0, 16)] = votes_v[pl.ds(k0, 16)] >> 2
                return 0

            lax.fori_loop(0, CHUNK // 16, shift_body, 0, unroll=4)
            cpy = pltpu.make_async_copy(yes_hbm.at[idx_v], yrows_v, sem)
            cpn = pltpu.make_async_copy(no_hbm.at[idx_v], nrows_v, sem)
            cpy.start()
            cpn.start()
            pltpu.sync_copy(legs_hbm.at[pl.ds(base, CHUNK)], legs_v)
            cpy.wait()
            cpn.wait()

            def group_body(i, _):
                k0 = i * 16
                v = votes_v[pl.ds(k0, 16)]
                q2 = (v & 3) * 2
                lv = legs_v[pl.ds(k0, 16)]
                g0 = plsc.load_gather(ip_v, [lv * 2])
                g1 = plsc.load_gather(ip_v, [lv * 2 + 1])
                ridx = lanes + k0
                yp0 = plsc.load_gather(yrows_v, [ridx, q2])
                yp1 = plsc.load_gather(yrows_v, [ridx, q2 + 1])
                np0 = plsc.load_gather(nrows_v, [ridx, q2])
                np1 = plsc.load_gather(nrows_v, [ridx, q2 + 1])
                dy0 = g0 - yp0
                dy1 = g1 - yp1
                dn0 = g0 - np0
                dn1 = g1 - np1
                dy = w20 * (dy0 * dy0) + w21 * (dy1 * dy1)
                dn = w20 * (dn0 * dn0) + w21 * (dn1 * dn1)
                out_v[pl.ds(k0, 16)] = jnp.exp(-0.5 * dy) - jnp.exp(-0.5 * dn)
                return 0

            lax.fori_loop(0, CHUNK // 16, group_body, 0, unroll=4)
            pltpu.sync_copy(out_v, out_hbm.at[pl.ds(base, CHUNK)])
            return 0

        lax.fori_loop(0, n_chunks, chunk_body, 0)

    return impl


def kernel(legs, votes, ideal_points, yes_points, no_points, w):
    yes8 = yes_points.reshape(V // 4, 8)
    no8 = no_points.reshape(V // 4, 8)
    wpad = jnp.pad(w.astype(jnp.float32), (0, 14))
    ip_flat = jnp.reshape(ideal_points.astype(jnp.float32), (-1,))
    info = plsc.get_sparse_core_info()
    nw = info.num_cores * info.num_subcores
    impl = _make_impl(B // nw)
    return impl(legs.astype(jnp.int32), votes.astype(jnp.int32),
                ip_flat, yes8, no8, wpad)
